# Initial kernel scaffold; baseline (speedup 1.0000x reference)
import jax
import jax.numpy as jnp
from jax.experimental import pallas as pl


def kernel(edge_src, edge_dst, edge_weight_cutoff, edge_attr, node_feat, Wk1, Wk2, Wk3, Wlog, Wv1, Wv2, Wv3, Wout):
    raise NotImplementedError("write your pallas kernel here")


# speedup vs baseline: 4.5134x; 4.5134x over previous
import jax
import jax.numpy as jnp
from jax import lax
from jax.experimental import pallas as pl
from jax.experimental.pallas import tpu as pltpu
from jax.experimental.pallas import tpu_sc as plsc

N = 10000
E = 320000
D = 128
H = 4
NC, NS = 2, 16
NW = NC * NS
EPW = E // NW
GB = 80
NG = EPW // GB
NPAD = 10240
NPT = NPAD // NS
ZR = NPAD * H // D
EB = 512
L = 16


def _silu(x):
    return x * jax.nn.sigmoid(x)


def _gather_body(nf, src3d, dst3d, xs_out, xd_out, idx_s, idx_d, rows, sem):
    c = lax.axis_index("c")
    s = lax.axis_index("s")
    w = c * NS + s
    pltpu.sync_copy(src3d.at[w], idx_s)
    pltpu.sync_copy(dst3d.at[w], idx_d)

    @pl.loop(0, NG)
    def _(i):
        base = w * EPW + i * GB
        pltpu.async_copy(nf.at[idx_s.at[i]], rows, sem).wait()
        pltpu.sync_copy(rows, xs_out.at[pl.ds(base, GB)])
        pltpu.async_copy(nf.at[idx_d.at[i]], rows, sem).wait()
        pltpu.sync_copy(rows, xd_out.at[pl.ds(base, GB)])


def _gather(node_feat, src3d, dst3d):
    mesh = plsc.VectorSubcoreMesh(core_axis_name="c", subcore_axis_name="s")
    return pl.kernel(
        _gather_body,
        out_type=(
            jax.ShapeDtypeStruct((E, D), jnp.float32),
            jax.ShapeDtypeStruct((E, D), jnp.float32),
        ),
        mesh=mesh,
        scratch_types=[
            pltpu.VMEM((NG, GB), jnp.int32),
            pltpu.VMEM((NG, GB), jnp.int32),
            pltpu.VMEM((GB, D), jnp.float32),
            pltpu.SemaphoreType.DMA,
        ],
    )(node_feat, src3d, dst3d)


def _edge_tc(xs, xd, attr, cw, wk1, wk2, wk3, wlt, wv1, wv2, wv3, out_v, out_w):
    sc = attr[...][:, 0:1]
    xsv = xs[...]
    xdv = xd[...]
    cwv = cw[...].reshape(EB, 1)

    h1 = _silu(sc * wk1[...])
    h2 = _silu(jnp.dot(h1, wk2[...], preferred_element_type=jnp.float32) * 0.125)
    km = jnp.dot(h2, wk3[...], preferred_element_type=jnp.float32) * 0.125
    key = xsv * km

    g1 = _silu(sc * wv1[...])
    g2 = _silu(jnp.dot(g1, wv2[...], preferred_element_type=jnp.float32) * 0.125)
    vm = jnp.dot(g2, wv3[...], preferred_element_type=jnp.float32) * 0.125

    scales = []
    ws = []
    for hh in range(H):
        tmp = jnp.dot(key, wlt[hh], preferred_element_type=jnp.float32)
        lg = jnp.sum(xdv * tmp, axis=1, keepdims=True) * (1.0 / D)
        w_h = cwv * jnp.exp(lg)
        ws.append(w_h)
        scales.append(jnp.broadcast_to(jnp.sqrt(w_h), (EB, D // H)))

    out_v[...] = xsv * vm * jnp.concatenate(scales, axis=1)
    out_w[...] = jnp.concatenate(ws, axis=1)


def _edge(xs, xd, edge_attr, cutoff, Wk1, Wk2, Wk3, wlt, Wv1, Wv2, Wv3):
    full = lambda *shape: pl.BlockSpec(shape, lambda i: (0,) * len(shape))
    return pl.pallas_call(
        _edge_tc,
        grid=(E // EB,),
        in_specs=[
            pl.BlockSpec((EB, D), lambda i: (i, 0)),
            pl.BlockSpec((EB, D), lambda i: (i, 0)),
            pl.BlockSpec((EB, 4), lambda i: (i, 0)),
            pl.BlockSpec((EB,), lambda i: (i,)),
            full(1, 64),
            full(64, 64),
            full(64, D),
            full(H, D, D),
            full(1, 64),
            full(64, 64),
            full(64, D),
        ],
        out_specs=[
            pl.BlockSpec((EB, D), lambda i: (i, 0)),
            pl.BlockSpec((EB, H), lambda i: (i, 0)),
        ],
        out_shape=[
            jax.ShapeDtypeStruct((E, D), jnp.float32),
            jax.ShapeDtypeStruct((E, H), jnp.float32),
        ],
        compiler_params=pltpu.CompilerParams(dimension_semantics=("arbitrary",)),
    )(xs, xd, edge_attr, cutoff, Wk1, Wk2, Wk3, wlt, Wv1, Wv2, Wv3)


def _scatter_val_body(ev, dst3d, zeros, parts, idx, rows, acc, sem):
    c = lax.axis_index("c")
    s = lax.axis_index("s")
    w = c * NS + s
    pltpu.sync_copy(zeros, acc.at[pl.ds(s * NPT, NPT)])
    pltpu.sync_copy(dst3d.at[w], idx)
    plsc.subcore_barrier()

    @pl.loop(0, NG)
    def _(i):
        base = w * EPW + i * GB
        pltpu.sync_copy(ev.at[pl.ds(base, GB)], rows)
        pltpu.sync_copy(rows, acc.at[idx.at[i]], add=True)

    plsc.subcore_barrier()
    pltpu.sync_copy(acc.at[pl.ds(s * NPT, NPT)],
                    parts.at[pl.ds(c * NPAD + s * NPT, NPT)])


def _scatter_val(ev, dst3d, zeros):
    mesh = plsc.VectorSubcoreMesh(core_axis_name="c", subcore_axis_name="s")
    return pl.kernel(
        _scatter_val_body,
        out_type=jax.ShapeDtypeStruct((NC * NPAD, D), jnp.float32),
        mesh=mesh,
        scratch_types=[
            pltpu.VMEM((NG, GB), jnp.int32),
            pltpu.VMEM((GB, D), jnp.float32),
            pltpu.VMEM_SHARED((NPAD, D), jnp.float32),
            pltpu.SemaphoreType.DMA,
        ],
    )(ev, dst3d, zeros)


def _scatter_z_body(w4, dst3d, zeros, iot2d, zout,
                    idx, wbuf, zacc, iotv, zsh, sem):
    c = lax.axis_index("c")
    s = lax.axis_index("s")
    w = c * NS + s

    @pl.when(s < ZR // 64)
    def _():
        pltpu.sync_copy(zeros.at[pl.ds(0, 64)], zsh.at[pl.ds(s * 64, 64)])

    pltpu.sync_copy(dst3d.at[w], idx)
    pltpu.sync_copy(zeros.at[pl.ds(0, ZR)], zacc)
    pltpu.sync_copy(iot2d, iotv)
    plsc.subcore_barrier()

    @pl.loop(0, NG)
    def _(i):
        base = w * EPW + i * GB
        pltpu.sync_copy(w4.at[pl.ds(base, GB)], wbuf)
        for j in range(GB // L):
            e16 = lax.iota(jnp.int32, L) + (j * L)
            dd = idx[i, pl.ds(j * L, L)]
            for hh in range(H):
                h16 = jnp.full((L,), hh, jnp.int32)
                wv = plsc.load_gather(wbuf, [e16, h16])
                a = dd * H + hh
                plsc.addupdate_scatter(
                    zacc, [lax.shift_right_logical(a, 7),
                           lax.bitwise_and(a, 127)], wv)

    for j in range(ZR // 64):
        pltpu.sync_copy(zacc.at[pl.ds(j * 64, 64)], zsh.at[iotv.at[j]], add=True)
    plsc.subcore_barrier()

    @pl.when(s == 0)
    def _():
        pltpu.sync_copy(zsh, zout.at[pl.ds(c * ZR, ZR)])


def _scatter_z(w4, dst3d, zeros, iot2d):
    mesh = plsc.VectorSubcoreMesh(core_axis_name="c", subcore_axis_name="s")
    return pl.kernel(
        _scatter_z_body,
        compiler_params=pltpu.CompilerParams(needs_layout_passes=False),
        out_type=jax.ShapeDtypeStruct((NC * ZR, D), jnp.float32),
        mesh=mesh,
        scratch_types=[
            pltpu.VMEM((NG, GB), jnp.int32),
            pltpu.VMEM((GB, H), jnp.float32),
            pltpu.VMEM((ZR, D), jnp.float32),
            pltpu.VMEM((ZR // 64, 64), jnp.int32),
            pltpu.VMEM_SHARED((ZR, D), jnp.float32),
            pltpu.SemaphoreType.DMA,
        ],
    )(w4, dst3d, zeros, iot2d)


NB = 1280


def _final_tc(p, zp, wout, out):
    pv = p[...]
    val = pv[0] + pv[1]
    zb = zp[...]
    z = zb[0] + zb[1]
    g = jnp.where(z > 0, lax.rsqrt(z), 1.0)
    gs = jnp.concatenate(
        [jnp.broadcast_to(g[:, hh:hh + 1], (NB, D // H)) for hh in range(H)], axis=1)
    out[...] = jnp.dot(val * gs, wout[...],
                       preferred_element_type=jnp.float32) * (D ** -0.5)


def _final(parts, z4, Wout):
    return pl.pallas_call(
        _final_tc,
        grid=(NPAD // NB,),
        in_specs=[
            pl.BlockSpec((NC, NB, D), lambda i: (0, i, 0)),
            pl.BlockSpec((NC, NB, H), lambda i: (0, i, 0)),
            pl.BlockSpec((D, D), lambda i: (0, 0)),
        ],
        out_specs=pl.BlockSpec((NB, D), lambda i: (i, 0)),
        out_shape=jax.ShapeDtypeStruct((NPAD, D), jnp.float32),
        compiler_params=pltpu.CompilerParams(dimension_semantics=("arbitrary",)),
    )(parts, z4, Wout)


def kernel(edge_src, edge_dst, edge_weight_cutoff, edge_attr, node_feat,
           Wk1, Wk2, Wk3, Wlog, Wv1, Wv2, Wv3, Wout):
    src3d = edge_src.reshape(NW, NG, GB)
    dst3d = edge_dst.reshape(NW, NG, GB)
    wlt = jnp.transpose(Wlog, (2, 1, 0))
    zeros = jnp.zeros((NPT, D), jnp.float32)
    iot2d = jnp.arange(ZR, dtype=jnp.int32).reshape(ZR // 64, 64)

    xs, xd = _gather(node_feat, src3d, dst3d)
    ev, w4 = _edge(xs, xd, edge_attr, edge_weight_cutoff,
                   Wk1, Wk2, Wk3, wlt, Wv1, Wv2, Wv3)
    parts = _scatter_val(ev, dst3d, zeros)
    zflat = _scatter_z(w4, dst3d, zeros, iot2d)
    z4 = zflat.reshape(NC, NPAD, H)
    out = _final(parts.reshape(NC, NPAD, D), z4, Wout)
    return out[:N]


# speedup vs baseline: 4.9833x; 1.1041x over previous
import jax
import jax.numpy as jnp
from jax import lax
from jax.experimental import pallas as pl
from jax.experimental.pallas import tpu as pltpu
from jax.experimental.pallas import tpu_sc as plsc

N = 10000
E = 320000
D = 128
H = 4
NC, NS = 2, 16
NW = NC * NS
EPW = E // NW
GB = 80
NG = EPW // GB
NPAD = 10240
NPT = NPAD // NS
ZR = NPAD * H // D
EB = 512
L = 16


def _silu(x):
    return x * jax.nn.sigmoid(x)


def _gather_body(nf, src3d, dst3d, xs_out, xd_out,
                 idx_s, idx_d, ra, rb, rc, rd, sa, sb, sc_, sd):
    c = lax.axis_index("c")
    s = lax.axis_index("s")
    w = c * NS + s
    pltpu.sync_copy(src3d.at[w], idx_s)
    pltpu.sync_copy(dst3d.at[w], idx_d)

    def fire_s(i, buf, sem):
        pltpu.async_copy(nf.at[idx_s.at[i]], buf, sem)

    def fire_d(i, buf, sem):
        pltpu.async_copy(nf.at[idx_d.at[i]], buf, sem)

    def drain(buf, sem):
        pltpu.make_async_copy(nf.at[idx_s.at[0]], buf, sem).wait()

    base0 = w * EPW
    fire_s(0, ra, sa)
    fire_d(0, rc, sc_)

    @pl.loop(0, NG - 1, step=2)
    def _(i):
        base = base0 + i * GB
        fire_s(i + 1, rb, sb)
        fire_d(i + 1, rd, sd)
        drain(ra, sa)
        pltpu.sync_copy(ra, xs_out.at[pl.ds(base, GB)])
        drain(rc, sc_)
        pltpu.sync_copy(rc, xd_out.at[pl.ds(base, GB)])
        fire_s(i + 2, ra, sa)
        fire_d(i + 2, rc, sc_)
        drain(rb, sb)
        pltpu.sync_copy(rb, xs_out.at[pl.ds(base + GB, GB)])
        drain(rd, sd)
        pltpu.sync_copy(rd, xd_out.at[pl.ds(base + GB, GB)])

    drain(ra, sa)
    pltpu.sync_copy(ra, xs_out.at[pl.ds(base0 + (NG - 1) * GB, GB)])
    drain(rc, sc_)
    pltpu.sync_copy(rc, xd_out.at[pl.ds(base0 + (NG - 1) * GB, GB)])


def _gather(node_feat, src3d, dst3d):
    mesh = plsc.VectorSubcoreMesh(core_axis_name="c", subcore_axis_name="s")
    return pl.kernel(
        _gather_body,
        out_type=(
            jax.ShapeDtypeStruct((E, D), jnp.float32),
            jax.ShapeDtypeStruct((E, D), jnp.float32),
        ),
        mesh=mesh,
        scratch_types=[
            pltpu.VMEM((NG, GB), jnp.int32),
            pltpu.VMEM((NG, GB), jnp.int32),
            pltpu.VMEM((GB, D), jnp.float32),
            pltpu.VMEM((GB, D), jnp.float32),
            pltpu.VMEM((GB, D), jnp.float32),
            pltpu.VMEM((GB, D), jnp.float32),
            pltpu.SemaphoreType.DMA,
            pltpu.SemaphoreType.DMA,
            pltpu.SemaphoreType.DMA,
            pltpu.SemaphoreType.DMA,
        ],
    )(node_feat, src3d, dst3d)


def _edge_tc(xs, xd, attr, cw, wk1, wk2, wk3, wlt, wv1, wv2, wv3, out_v, out_w):
    sc = attr[...][:, 0:1]
    xsv = xs[...]
    xdv = xd[...]
    cwv = cw[...].reshape(EB, 1)

    h1 = _silu(sc * wk1[...])
    h2 = _silu(jnp.dot(h1, wk2[...], preferred_element_type=jnp.float32) * 0.125)
    km = jnp.dot(h2, wk3[...], preferred_element_type=jnp.float32) * 0.125
    key = xsv * km

    g1 = _silu(sc * wv1[...])
    g2 = _silu(jnp.dot(g1, wv2[...], preferred_element_type=jnp.float32) * 0.125)
    vm = jnp.dot(g2, wv3[...], preferred_element_type=jnp.float32) * 0.125

    tmp = jnp.dot(key.astype(jnp.bfloat16), wlt[...],
                  preferred_element_type=jnp.float32)
    scales = []
    ws = []
    for hh in range(H):
        lg = jnp.sum(xdv * tmp[:, hh * D:(hh + 1) * D], axis=1,
                     keepdims=True) * (1.0 / D)
        w_h = cwv * jnp.exp(lg)
        ws.append(w_h)
        scales.append(jnp.broadcast_to(jnp.sqrt(w_h), (EB, D // H)))

    out_v[...] = xsv * vm * jnp.concatenate(scales, axis=1)
    out_w[...] = jnp.concatenate(ws, axis=1)


def _edge(xs, xd, edge_attr, cutoff, Wk1, Wk2, Wk3, wlt, Wv1, Wv2, Wv3):
    full = lambda *shape: pl.BlockSpec(shape, lambda i: (0,) * len(shape))
    return pl.pallas_call(
        _edge_tc,
        grid=(E // EB,),
        in_specs=[
            pl.BlockSpec((EB, D), lambda i: (i, 0)),
            pl.BlockSpec((EB, D), lambda i: (i, 0)),
            pl.BlockSpec((EB, 4), lambda i: (i, 0)),
            pl.BlockSpec((EB,), lambda i: (i,)),
            full(1, 64),
            full(64, 64),
            full(64, D),
            full(D, H * D),
            full(1, 64),
            full(64, 64),
            full(64, D),
        ],
        out_specs=[
            pl.BlockSpec((EB, D), lambda i: (i, 0)),
            pl.BlockSpec((EB, H), lambda i: (i, 0)),
        ],
        out_shape=[
            jax.ShapeDtypeStruct((E, D), jnp.float32),
            jax.ShapeDtypeStruct((E, H), jnp.float32),
        ],
        compiler_params=pltpu.CompilerParams(dimension_semantics=("arbitrary",)),
    )(xs, xd, edge_attr, cutoff, Wk1, Wk2, Wk3, wlt, Wv1, Wv2, Wv3)


def _scatter_val_body(ev, dst3d, zeros, parts, idx, rows, acc, sem):
    c = lax.axis_index("c")
    s = lax.axis_index("s")
    w = c * NS + s
    pltpu.sync_copy(zeros, acc.at[pl.ds(s * NPT, NPT)])
    pltpu.sync_copy(dst3d.at[w], idx)
    plsc.subcore_barrier()

    @pl.loop(0, NG)
    def _(i):
        base = w * EPW + i * GB
        pltpu.sync_copy(ev.at[pl.ds(base, GB)], rows)
        pltpu.sync_copy(rows, acc.at[idx.at[i]], add=True)

    plsc.subcore_barrier()
    pltpu.sync_copy(acc.at[pl.ds(s * NPT, NPT)],
                    parts.at[pl.ds(c * NPAD + s * NPT, NPT)])


def _scatter_val(ev, dst3d, zeros):
    mesh = plsc.VectorSubcoreMesh(core_axis_name="c", subcore_axis_name="s")
    return pl.kernel(
        _scatter_val_body,
        out_type=jax.ShapeDtypeStruct((NC * NPAD, D), jnp.float32),
        mesh=mesh,
        scratch_types=[
            pltpu.VMEM((NG, GB), jnp.int32),
            pltpu.VMEM((GB, D), jnp.float32),
            pltpu.VMEM_SHARED((NPAD, D), jnp.float32),
            pltpu.SemaphoreType.DMA,
        ],
    )(ev, dst3d, zeros)


def _scatter_z_body(w4, dst3d, zeros, iot2d, zout,
                    idx, wbuf, zacc, iotv, zsh, sem):
    c = lax.axis_index("c")
    s = lax.axis_index("s")
    w = c * NS + s

    @pl.when(s < ZR // 64)
    def _():
        pltpu.sync_copy(zeros.at[pl.ds(0, 64)], zsh.at[pl.ds(s * 64, 64)])

    pltpu.sync_copy(dst3d.at[w], idx)
    pltpu.sync_copy(zeros.at[pl.ds(0, ZR)], zacc)
    pltpu.sync_copy(iot2d, iotv)
    plsc.subcore_barrier()

    @pl.loop(0, NG)
    def _(i):
        base = w * EPW + i * GB
        pltpu.sync_copy(w4.at[pl.ds(base, GB)], wbuf)
        for j in range(GB // L):
            e16 = lax.iota(jnp.int32, L) + (j * L)
            dd = idx[i, pl.ds(j * L, L)]
            for hh in range(H):
                h16 = jnp.full((L,), hh, jnp.int32)
                wv = plsc.load_gather(wbuf, [e16, h16])
                a = dd * H + hh
                plsc.addupdate_scatter(
                    zacc, [lax.shift_right_logical(a, 7),
                           lax.bitwise_and(a, 127)], wv)

    for j in range(ZR // 64):
        pltpu.sync_copy(zacc.at[pl.ds(j * 64, 64)], zsh.at[iotv.at[j]], add=True)
    plsc.subcore_barrier()

    @pl.when(s == 0)
    def _():
        pltpu.sync_copy(zsh, zout.at[pl.ds(c * ZR, ZR)])


def _scatter_z(w4, dst3d, zeros, iot2d):
    mesh = plsc.VectorSubcoreMesh(core_axis_name="c", subcore_axis_name="s")
    return pl.kernel(
        _scatter_z_body,
        compiler_params=pltpu.CompilerParams(needs_layout_passes=False),
        out_type=jax.ShapeDtypeStruct((NC * ZR, D), jnp.float32),
        mesh=mesh,
        scratch_types=[
            pltpu.VMEM((NG, GB), jnp.int32),
            pltpu.VMEM((GB, H), jnp.float32),
            pltpu.VMEM((ZR, D), jnp.float32),
            pltpu.VMEM((ZR // 64, 64), jnp.int32),
            pltpu.VMEM_SHARED((ZR, D), jnp.float32),
            pltpu.SemaphoreType.DMA,
        ],
    )(w4, dst3d, zeros, iot2d)


NB = 1280


def _final_tc(p, zp, wout, out):
    pv = p[...]
    val = pv[0] + pv[1]
    zb = zp[...]
    z = zb[0] + zb[1]
    g = jnp.where(z > 0, lax.rsqrt(z), 1.0)
    gs = jnp.concatenate(
        [jnp.broadcast_to(g[:, hh:hh + 1], (NB, D // H)) for hh in range(H)], axis=1)
    out[...] = jnp.dot(val * gs, wout[...],
                       preferred_element_type=jnp.float32) * (D ** -0.5)


def _final(parts, z4, Wout):
    return pl.pallas_call(
        _final_tc,
        grid=(NPAD // NB,),
        in_specs=[
            pl.BlockSpec((NC, NB, D), lambda i: (0, i, 0)),
            pl.BlockSpec((NC, NB, H), lambda i: (0, i, 0)),
            pl.BlockSpec((D, D), lambda i: (0, 0)),
        ],
        out_specs=pl.BlockSpec((NB, D), lambda i: (i, 0)),
        out_shape=jax.ShapeDtypeStruct((NPAD, D), jnp.float32),
        compiler_params=pltpu.CompilerParams(dimension_semantics=("arbitrary",)),
    )(parts, z4, Wout)


def kernel(edge_src, edge_dst, edge_weight_cutoff, edge_attr, node_feat,
           Wk1, Wk2, Wk3, Wlog, Wv1, Wv2, Wv3, Wout):
    src3d = edge_src.reshape(NW, NG, GB)
    dst3d = edge_dst.reshape(NW, NG, GB)
    wlt = jnp.transpose(Wlog, (1, 2, 0)).reshape(D, H * D).astype(jnp.bfloat16)
    zeros = jnp.zeros((NPT, D), jnp.float32)
    iot2d = jnp.arange(ZR, dtype=jnp.int32).reshape(ZR // 64, 64)

    xs, xd = _gather(node_feat, src3d, dst3d)
    ev, w4 = _edge(xs, xd, edge_attr, edge_weight_cutoff,
                   Wk1, Wk2, Wk3, wlt, Wv1, Wv2, Wv3)
    parts = _scatter_val(ev, dst3d, zeros)
    zflat = _scatter_z(w4, dst3d, zeros, iot2d)
    z4 = zflat.reshape(NC, NPAD, H)
    out = _final(parts.reshape(NC, NPAD, D), z4, Wout)
    return out[:N]
